# Initial kernel scaffold; baseline (speedup 1.0000x reference)
import jax
import jax.numpy as jnp
from jax.experimental import pallas as pl


def kernel(input_vector, weights):
    raise NotImplementedError("write your pallas kernel here")


# speedup vs baseline: 3.4969x; 3.4969x over previous
import jax
import jax.numpy as jnp
from jax.experimental import pallas as pl

_M, _N, _DIM = 64, 64, 16
_K = _M * _N
_BQ = 256


def _bmu_kernel(x_ref, wt_ref, idx_ref):
    acc = (wt_ref[0:1, :] - x_ref[:, 0:1]) ** 2
    for d in range(1, _DIM):
        acc += (wt_ref[d : d + 1, :] - x_ref[:, d : d + 1]) ** 2
    m = jnp.min(acc, axis=1, keepdims=True)
    iota = jax.lax.broadcasted_iota(jnp.int32, (_BQ, _K), 1)
    idx = jnp.min(jnp.where(acc == m, iota, _K), axis=1, keepdims=True)
    idx_ref[...] = idx


def kernel(input_vector, weights):
    b = input_vector.shape[0]
    wt = weights.T
    idx = pl.pallas_call(
        _bmu_kernel,
        grid=(b // _BQ,),
        in_specs=[
            pl.BlockSpec((_BQ, _DIM), lambda i: (i, 0)),
            pl.BlockSpec((_DIM, _K), lambda i: (0, 0)),
        ],
        out_specs=pl.BlockSpec((_BQ, 1), lambda i: (i, 0)),
        out_shape=jax.ShapeDtypeStruct((b, 1), jnp.int32),
    )(input_vector, wt)
    idx = idx[:, 0]
    return jnp.stack([idx // _N, idx % _N], axis=-1)


# speedup vs baseline: 7.3092x; 2.0902x over previous
import jax
import jax.numpy as jnp
from jax.experimental import pallas as pl

_M, _N, _DIM = 64, 64, 16
_K = _M * _N
_BQ = 256


def _bmu_kernel(x_ref, wt_ref, w_ref, idx_ref):
    x = x_ref[...]
    wt = wt_ref[...]
    wn = jnp.sum(wt * wt, axis=0, keepdims=True)
    g = jnp.dot(x, wt, preferred_element_type=jnp.float32)
    s = g - 0.5 * wn
    iota = jax.lax.broadcasted_iota(jnp.int32, (_BQ, _K), 1)
    m1 = jnp.max(s, axis=1, keepdims=True)
    i1 = jnp.min(jnp.where(s == m1, iota, _K), axis=1, keepdims=True)
    s2 = jnp.where(iota == i1, -jnp.inf, s)
    m2 = jnp.max(s2, axis=1, keepdims=True)
    i2 = jnp.min(jnp.where(s2 == m2, iota, _K), axis=1, keepdims=True)
    oh1 = (iota == i1).astype(jnp.float32)
    oh2 = (iota == i2).astype(jnp.float32)
    r1 = jnp.dot(oh1, w_ref[...], preferred_element_type=jnp.float32)
    r2 = jnp.dot(oh2, w_ref[...], preferred_element_type=jnp.float32)
    d1 = jnp.sum((r1 - x) ** 2, axis=1, keepdims=True)
    d2 = jnp.sum((r2 - x) ** 2, axis=1, keepdims=True)
    take2 = (d2 < d1) | ((d2 == d1) & (i2 < i1))
    idx_ref[...] = jnp.where(take2, i2, i1)


def kernel(input_vector, weights):
    b = input_vector.shape[0]
    wt = weights.T
    idx = pl.pallas_call(
        _bmu_kernel,
        grid=(b // _BQ,),
        in_specs=[
            pl.BlockSpec((_BQ, _DIM), lambda i: (i, 0)),
            pl.BlockSpec((_DIM, _K), lambda i: (0, 0)),
            pl.BlockSpec((_K, _DIM), lambda i: (0, 0)),
        ],
        out_specs=pl.BlockSpec((_BQ, 1), lambda i: (i, 0)),
        out_shape=jax.ShapeDtypeStruct((b, 1), jnp.int32),
    )(input_vector, wt, weights)
    idx = idx[:, 0]
    return jnp.stack([idx // _N, idx % _N], axis=-1)


# speedup vs baseline: 8.5858x; 1.1746x over previous
import jax
import jax.numpy as jnp
from jax.experimental import pallas as pl

_M, _N, _DIM = 64, 64, 16
_K = _M * _N
_BQ = 1024


def _bmu_kernel(x_ref, wt_ref, idx_ref):
    x = x_ref[...]
    wt = wt_ref[...]
    wn = jnp.sum(wt * wt, axis=0, keepdims=True)
    bf, f32 = jnp.bfloat16, jnp.float32
    xh = x.astype(bf)
    xr = x - xh.astype(f32)
    xm = xr.astype(bf)
    xl = (xr - xm.astype(f32)).astype(bf)
    wh = wt.astype(bf)
    wr = wt - wh.astype(f32)
    wm = wr.astype(bf)
    wl = (wr - wm.astype(f32)).astype(bf)
    xa = jnp.concatenate([xh, xh, xm, xh, xl, xm], axis=1)
    wa = jnp.concatenate([wh, wm, wh, wl, wh, wm], axis=0)
    g = jnp.dot(xa, wa, preferred_element_type=jnp.float32)
    s = g - 0.5 * wn
    iota = jax.lax.broadcasted_iota(jnp.int32, (_BQ, _K), 1)
    i1 = jnp.argmax(s, axis=1, keepdims=True).astype(jnp.int32)
    s2 = jnp.where(iota == i1, -jnp.inf, s)
    i2 = jnp.argmax(s2, axis=1, keepdims=True).astype(jnp.int32)
    oh = jnp.concatenate(
        [(iota == i1).astype(jnp.float32), (iota == i2).astype(jnp.float32)],
        axis=0,
    )
    wsplit = jnp.concatenate(
        [wh.astype(f32), wm.astype(f32), wl.astype(f32)], axis=0
    )
    rs = jax.lax.dot_general(
        oh, wsplit, (((1,), (1,)), ((), ())),
        preferred_element_type=jnp.float32,
    )
    eye = (
        jax.lax.broadcasted_iota(jnp.int32, (3 * _DIM, _DIM), 0) % _DIM
        == jax.lax.broadcasted_iota(jnp.int32, (3 * _DIM, _DIM), 1)
    ).astype(jnp.float32)
    rr = jnp.dot(rs, eye, preferred_element_type=jnp.float32)
    r1, r2 = rr[:_BQ], rr[_BQ:]
    d1 = jnp.sum((r1 - x) ** 2, axis=1, keepdims=True)
    d2 = jnp.sum((r2 - x) ** 2, axis=1, keepdims=True)
    take2 = (d2 < d1) | ((d2 == d1) & (i2 < i1))
    best = jnp.where(take2, i2, i1)
    idx_ref[...] = jnp.concatenate([best >> 6, best & 63], axis=1)


def kernel(input_vector, weights):
    b = input_vector.shape[0]
    wt = weights.T
    idx = pl.pallas_call(
        _bmu_kernel,
        grid=(b // _BQ,),
        in_specs=[
            pl.BlockSpec((_BQ, _DIM), lambda i: (i, 0)),
            pl.BlockSpec((_DIM, _K), lambda i: (0, 0)),
        ],
        out_specs=pl.BlockSpec((_BQ, 2), lambda i: (i, 0)),
        out_shape=jax.ShapeDtypeStruct((b, 2), jnp.int32),
    )(input_vector, wt)
    return idx
